# Initial kernel scaffold; baseline (speedup 1.0000x reference)
import jax
import jax.numpy as jnp
from jax.experimental import pallas as pl


def kernel(unseen_entity, triplets, entity_table, relation_embedding, basis, att):
    raise NotImplementedError("write your pallas kernel here")


# speedup vs baseline: 48.9058x; 48.9058x over previous
import functools

import jax
import jax.numpy as jnp
from jax import lax
from jax.experimental import pallas as pl
from jax.experimental.pallas import tpu as pltpu
from jax.experimental.pallas import tpu_sc as plsc

NREL = 200
ENT_DIM = 128
REL_DIM = 16
IN_CH = ENT_DIM + REL_DIM
NBASES = 4
N_TRI = 50000

NC, NS, L = 2, 16, 16
NW = NC * NS
CHUNK = 1568
NPAD = CHUNK * NW
NBLK = CHUNK // L
CAP = 2 * CHUNK + L
OUTW = 640


def _splat(vec, j):
    idx = jnp.full((L,), j, dtype=jnp.int32)
    return vec.at[idx].get(mode="promise_in_bounds")


def _sc_body(u_hbm, src_hbm, dst_hbm, rel_hbm, att_hbm, relemb_hbm, ent_hbm,
             out_hbm,
             src_ref, dst_ref, rel_ref, u_ref, att_ref, relemb_ref,
             gidx_ref, ridx_ref, c0_ref, c1_ref, c2_ref, c3_ref,
             rows_ref, stage_ref, sem):
    wid = lax.axis_index("s") * NC + lax.axis_index("c")
    base = wid * CHUNK
    pltpu.sync_copy(src_hbm.at[pl.ds(base, CHUNK)], src_ref)
    pltpu.sync_copy(dst_hbm.at[pl.ds(base, CHUNK)], dst_ref)
    pltpu.sync_copy(rel_hbm.at[pl.ds(base, CHUNK)], rel_ref)
    pltpu.sync_copy(att_hbm, att_ref)
    pltpu.sync_copy(relemb_hbm, relemb_ref)
    pltpu.sync_copy(u_hbm, u_ref)

    uv = u_ref[...]
    iota = lax.broadcasted_iota(jnp.int32, (L,), 0)
    crefs = (c0_ref, c1_ref, c2_ref, c3_ref)

    def scan_blk(b, nv):
        off = b * L
        sv = src_ref[pl.ds(off, L)]
        dv = dst_ref[pl.ds(off, L)]
        m1 = dv == uv
        m2 = sv == uv
        mi1 = m1.astype(jnp.int32)
        mi2 = m2.astype(jnp.int32)
        incl1 = plsc.cumsum(mi1)
        incl2 = plsc.cumsum(mi2)
        pc1 = _splat(incl1, L - 1)
        pc2 = _splat(incl2, L - 1)

        @pl.when(jnp.any(m1 | m2))
        def _():
            rv = rel_ref[pl.ds(off, L)]

            def emit(nbase, mask, incl, mi, ent_idx, att_row):
                pos = nbase + incl - mi
                plsc.store_scatter(gidx_ref, [pos], ent_idx, mask=mask)
                plsc.store_scatter(ridx_ref, [pos], rv, mask=mask)
                for bb in range(NBASES):
                    cb = plsc.load_gather(att_ref, [att_row * NBASES + bb])
                    plsc.store_scatter(crefs[bb], [pos], cb, mask=mask)

            emit(nv, m1, incl1, mi1, sv, rv)
            emit(nv + pc1, m2, incl2, mi2, dv, rv + NREL)

        return nv + pc1 + pc2

    nv = lax.fori_loop(0, NBLK, scan_blk, jnp.zeros((L,), jnp.int32))
    n = jnp.max(nv)

    zpos = nv + iota
    zi = jnp.zeros((L,), jnp.int32)
    zf = jnp.zeros((L,), jnp.float32)
    plsc.store_scatter(gidx_ref, [zpos], zi)
    plsc.store_scatter(ridx_ref, [zpos], zi)
    for cref in crefs:
        plsc.store_scatter(cref, [zpos], zf)

    nblocks = (n + L - 1) // L

    def match_blk(jb, accs):
        o = jb * L
        idxv = gidx_ref[pl.ds(o, L)]
        cp = pltpu.async_copy(ent_hbm.at[idxv], rows_ref, sem)
        rix = ridx_ref[pl.ds(o, L)]
        cs = [cref[pl.ds(o, L)] for cref in crefs]
        cp.wait()
        accs = list(accs)
        for j in range(L):
            csp = [_splat(cs[bb], j) for bb in range(NBASES)]
            rsp = _splat(rix, j)
            relvec = plsc.load_gather(relemb_ref, [rsp * REL_DIM + iota])
            for k in range(ENT_DIM // L):
                fv = rows_ref[j, pl.ds(k * L, L)]
                for bb in range(NBASES):
                    accs[bb * 9 + k] = accs[bb * 9 + k] + csp[bb] * fv
            for bb in range(NBASES):
                accs[bb * 9 + 8] = accs[bb * 9 + 8] + csp[bb] * relvec
        return tuple(accs)

    acc0 = tuple(zf for _ in range(NBASES * 9))
    accs = lax.fori_loop(0, nblocks, match_blk, acc0)

    for bb in range(NBASES):
        for k in range(9):
            stage_ref[pl.ds(bb * IN_CH + k * L, L)] = accs[bb * 9 + k]
    cntf = nv.astype(jnp.float32)
    stage_ref[pl.ds(NBASES * IN_CH, L)] = jnp.where(iota == 0, cntf, zf)
    for k in range(NBASES * IN_CH + L, OUTW, L):
        stage_ref[pl.ds(k, L)] = zf
    pltpu.sync_copy(stage_ref, out_hbm.at[wid])


def _tc_body(part_ref, basis_ref, out_ref):
    part = part_ref[...]
    prod = jnp.dot(part, basis_ref[...],
                   preferred_element_type=jnp.float32)
    s = jnp.sum(prod, axis=0, keepdims=True)
    col = lax.broadcasted_iota(jnp.int32, part.shape, 1)
    cnt = jnp.sum(jnp.where(col == NBASES * IN_CH, part, 0.0))
    denom = jnp.maximum(cnt, 1.0)
    out_ref[...] = jnp.maximum(s / denom, 0.0)


@jax.jit
def kernel(unseen_entity, triplets, entity_table, relation_embedding, basis,
           att):
    trip = jnp.asarray(triplets).astype(jnp.int32)
    pad = NPAD - N_TRI
    src = jnp.concatenate([trip[:, 0], jnp.full((pad,), -1, jnp.int32)])
    rel = jnp.concatenate([trip[:, 1], jnp.zeros((pad,), jnp.int32)])
    dst = jnp.concatenate([trip[:, 2], jnp.full((pad,), -1, jnp.int32)])
    u_arr = jnp.full((L,), jnp.asarray(unseen_entity, jnp.int32))
    att_flat = att.reshape(-1)
    relemb_flat = relation_embedding.reshape(-1)

    sc_fn = pl.kernel(
        _sc_body,
        out_type=jax.ShapeDtypeStruct((NW, OUTW), jnp.float32),
        mesh=plsc.VectorSubcoreMesh(core_axis_name="c", subcore_axis_name="s",
                                    num_cores=NC, num_subcores=NS),
        compiler_params=pltpu.CompilerParams(needs_layout_passes=False),
        scratch_types=[
            pltpu.VMEM((CHUNK,), jnp.int32),
            pltpu.VMEM((CHUNK,), jnp.int32),
            pltpu.VMEM((CHUNK,), jnp.int32),
            pltpu.VMEM((L,), jnp.int32),
            pltpu.VMEM((2 * NREL * NBASES,), jnp.float32),
            pltpu.VMEM((NREL * REL_DIM,), jnp.float32),
            pltpu.VMEM((CAP,), jnp.int32),
            pltpu.VMEM((CAP,), jnp.int32),
            pltpu.VMEM((CAP,), jnp.float32),
            pltpu.VMEM((CAP,), jnp.float32),
            pltpu.VMEM((CAP,), jnp.float32),
            pltpu.VMEM((CAP,), jnp.float32),
            pltpu.VMEM((L, ENT_DIM), jnp.float32),
            pltpu.VMEM((OUTW,), jnp.float32),
            pltpu.SemaphoreType.DMA,
        ],
    )
    partials = sc_fn(u_arr, src, dst, rel, att_flat, relemb_flat,
                     entity_table)

    basis_pad = jnp.zeros((OUTW, ENT_DIM), jnp.float32)
    basis_pad = basis_pad.at[:NBASES * IN_CH].set(
        basis.reshape(NBASES * IN_CH, ENT_DIM))

    out = pl.pallas_call(
        _tc_body,
        out_shape=jax.ShapeDtypeStruct((1, ENT_DIM), jnp.float32),
    )(partials, basis_pad)
    return out[0]
